# baseline (speedup 1.0000x reference)
import functools

import jax
import jax.numpy as jnp
from jax import lax
from jax.experimental import pallas as pl
from jax.experimental.pallas import tpu as pltpu
from jax.experimental.pallas import tpu_sc as plsc

_NUM_CORES = 2
_NUM_SUBCORES = 16
_NW = _NUM_CORES * _NUM_SUBCORES
_L = 16
_TPAD = 129
_BG = 16


def _gather_sc_t(idxT, table):
    h, nb = idxT.shape
    d = table.shape[1]
    bpw = nb // _NW
    assert d % _L == 0 and bpw % _BG == 0 and h % 2 == 0

    mesh = plsc.VectorSubcoreMesh(
        core_axis_name="c", subcore_axis_name="s",
        num_cores=_NUM_CORES, num_subcores=_NUM_SUBCORES,
    )

    @functools.partial(
        pl.kernel,
        mesh=mesh,
        compiler_params=pltpu.CompilerParams(use_tc_tiling_on_sc=False, needs_layout_passes=False),
        out_type=jax.ShapeDtypeStruct((h, d // 8, _NW, 8, 128), jnp.float32),
        scratch_types=[
            pltpu.VMEM((h, bpw), jnp.int32),
            pltpu.VMEM((bpw, d), jnp.float32),
            pltpu.VMEM((bpw, d), jnp.float32),
            pltpu.VMEM((d // 8, 8, _TPAD), jnp.float32),
            pltpu.VMEM((d // 8, 8, _TPAD), jnp.float32),
            pltpu.SemaphoreType.DMA((2,)),
            pltpu.SemaphoreType.DMA((2,)),
        ],
    )
    def k(idxT_hbm, table_hbm, out_hbm, idx_v, rows_v0, rows_v1, t_v0, t_v1, gsem, osem):
        wid = lax.axis_index("s") * _NUM_CORES + lax.axis_index("c")
        col0 = wid * bpw
        pltpu.sync_copy(idxT_hbm.at[pl.ds(0, h), pl.ds(col0, bpw)], idx_v)

        iota = lax.iota(jnp.int32, _L)

        def gather_cp(hh, r):
            return pltpu.make_async_copy(
                table_hbm.at[idx_v.at[hh]],
                rows_v0 if r == 0 else rows_v1,
                gsem.at[r],
            )

        def out_cp(hh, r):
            return pltpu.make_async_copy(
                (t_v0 if r == 0 else t_v1).at[pl.ds(0, d // 8), pl.ds(0, 8), pl.ds(0, 128)],
                out_hbm.at[hh, pl.ds(0, d // 8), wid],
                osem.at[r],
            )

        def transpose(r):
            rows = rows_v0 if r == 0 else rows_v1
            t = t_v0 if r == 0 else t_v1

            def tbody(g, carry):
                b0 = g * _L
                bidx = iota + b0
                for dd in range(d):
                    vec = plsc.load_gather(
                        rows, [bidx, jnp.full((_L,), dd, jnp.int32)]
                    )
                    t[dd // 8, dd % 8, pl.ds(b0, _L)] = vec
                return carry

            lax.fori_loop(0, bpw // _L, tbody, 0)

        def steady(hh, r, wait_prev_out, issue_next):
            if wait_prev_out:
                out_cp(hh - 2, r).wait()
            gather_cp(hh, r).wait()
            if issue_next:
                gather_cp(hh + 1, 1 - r).start()
            transpose(r)
            out_cp(hh, r).start()

        gather_cp(jnp.int32(0), 0).start()
        steady(jnp.int32(0), 0, wait_prev_out=False, issue_next=True)
        steady(jnp.int32(1), 1, wait_prev_out=False, issue_next=True)

        def body(q, carry):
            h0 = 2 + 2 * q
            steady(h0, 0, wait_prev_out=True, issue_next=True)
            steady(h0 + 1, 1, wait_prev_out=True, issue_next=True)
            return carry

        lax.fori_loop(0, (h - 4) // 2, body, 0)

        steady(jnp.int32(h - 2), 0, wait_prev_out=True, issue_next=True)
        steady(jnp.int32(h - 1), 1, wait_prev_out=True, issue_next=False)
        out_cp(jnp.int32(h - 2), 0).wait()
        out_cp(jnp.int32(h - 1), 1).wait()

    return k(idxT, table)


@jax.jit
def _embed(indices, table):
    nb, h = indices.shape
    d = table.shape[1]
    idxT = indices.T.astype(jnp.int32)
    y2 = _gather_sc_t(idxT, table)
    return y2.transpose((2, 4, 0, 1, 3)).reshape(nb, h, d)


def kernel(indices, table):
    return _embed(indices, table)


# speedup vs baseline: 3.2676x; 3.2676x over previous
import functools

import jax
import jax.numpy as jnp
from jax import lax
from jax.experimental import pallas as pl
from jax.experimental.pallas import tpu as pltpu
from jax.experimental.pallas import tpu_sc as plsc

_NUM_CORES = 2
_NUM_SUBCORES = 16
_NW = _NUM_CORES * _NUM_SUBCORES
_NBUF = 2
_BPC = 4


def _gather_sc(indices, table):
    nb, h = indices.shape
    d = table.shape[1]
    b_per_w = nb // _NW
    n_chunks = b_per_w // _BPC
    assert n_chunks % _NBUF == 0 and n_chunks >= 2 * _NBUF

    mesh = plsc.VectorSubcoreMesh(
        core_axis_name="c", subcore_axis_name="s",
        num_cores=_NUM_CORES, num_subcores=_NUM_SUBCORES,
    )

    @functools.partial(
        pl.kernel,
        mesh=mesh,
        compiler_params=pltpu.CompilerParams(use_tc_tiling_on_sc=False, needs_layout_passes=False),
        out_type=jax.ShapeDtypeStruct((nb, h, 2 * d), jnp.float32),
        scratch_types=[
            pltpu.VMEM((_NBUF, _BPC, h), jnp.int32),
            pltpu.VMEM((_NBUF, _BPC, h, d), jnp.float32),
            pltpu.SemaphoreType.DMA((_NBUF,)),
            pltpu.SemaphoreType.DMA((_NBUF,)),
        ],
    )
    def k(idx_hbm, table_hbm, out_hbm, idx_v, rows_v, gsem, osem):
        wid = lax.axis_index("s") * _NUM_CORES + lax.axis_index("c")
        base = wid * b_per_w

        def out_copies(bi, b, wait):
            for j in range(_BPC):
                cp = pltpu.make_async_copy(
                    rows_v.at[b, j],
                    out_hbm.at[bi + j, pl.ds(0, h), pl.ds(0, d)],
                    osem.at[b],
                )
                if wait:
                    cp.wait()
                else:
                    cp.start()

        def step(cc, b, wait_out):
            bi = base + cc * _BPC
            if wait_out:
                out_copies(bi, b, wait=True)
            pltpu.sync_copy(idx_hbm.at[pl.ds(bi, _BPC)], idx_v.at[b])
            for j in range(_BPC):
                pltpu.async_copy(
                    table_hbm.at[idx_v.at[b, j]], rows_v.at[b, j],
                    gsem.at[b],
                )
            for j in range(_BPC):
                pltpu.make_async_copy(
                    table_hbm.at[idx_v.at[b, j]], rows_v.at[b, j],
                    gsem.at[b],
                ).wait()
            out_copies(bi, b, wait=False)

        for b in range(_NBUF):
            step(jnp.int32(b), b, wait_out=False)

        def body(r, carry):
            c0 = _NBUF + r * _NBUF
            for b in range(_NBUF):
                step(c0 + b, b, wait_out=True)
            return carry

        lax.fori_loop(0, n_chunks // _NBUF - 1, body, 0)

        for b in range(_NBUF):
            bi = base + (n_chunks - _NBUF + b) * _BPC
            out_copies(bi, b, wait=True)

    return k(indices, table)


@jax.jit
def _embed(indices, table):
    wide = _gather_sc(indices.astype(jnp.int32), table)
    return wide[:, :, :table.shape[1]]


def kernel(indices, table):
    return _embed(indices, table)
